# baseline (speedup 1.0000x reference)
import jax
import jax.numpy as jnp
from jax.experimental import pallas as pl
from jax.experimental.pallas import tpu as pltpu

_NC = 1000
_CK = 200
_NBUF = 12


def _onehot_ring(xt_ref, o_ref, buf, sem):
    i = pl.program_id(0)
    nsteps = pl.num_programs(0)
    nchunk = _NC // _CK
    j = i // nchunk
    cb = jax.lax.rem(i, nchunk)
    slot = jax.lax.rem(i, _NBUF)

    @pl.when(i >= _NBUF)
    def _():
        pltpu.make_async_copy(
            buf.at[slot], o_ref.at[0, pl.ds(0, _CK), :], sem.at[slot]
        ).wait()

    @pl.when(i < _NBUF)
    def _():
        buf[slot] = jnp.zeros((_CK, o_ref.shape[2]), jnp.float32)
    pltpu.make_async_copy(
        buf.at[slot], o_ref.at[j, pl.ds(cb * _CK, _CK), :], sem.at[slot]
    ).start()

    @pl.when(i == nsteps - 1)
    def _():
        for k in range(_NBUF):
            pltpu.make_async_copy(
                buf.at[k], o_ref.at[0, pl.ds(0, _CK), :], sem.at[k]
            ).wait()


def kernel(x):
    B, S = x.shape
    nchunk = _NC // _CK
    xt = x.T.reshape(S, 1, B)
    yt = pl.pallas_call(
        _onehot_ring,
        grid=(S * nchunk,),
        in_specs=[pl.BlockSpec((1, 1, B), lambda i: (i // (_NC // _CK), 0, 0))],
        out_specs=pl.BlockSpec(memory_space=pl.ANY),
        out_shape=jax.ShapeDtypeStruct((S, _NC, B), jnp.float32),
        scratch_shapes=[
            pltpu.VMEM((_NBUF, _CK, B), jnp.float32),
            pltpu.SemaphoreType.DMA((_NBUF,)),
        ],
    )(xt)
    return yt.transpose(2, 0, 1)


# speedup vs baseline: 1.0107x; 1.0107x over previous
import jax
import jax.numpy as jnp
from jax.experimental import pallas as pl
from jax.experimental.pallas import tpu as pltpu

_NC = 1000
_BC = 1024
_NBUF = 8


def _onehot_ring(xt_ref, o_ref, buf, sem):
    i = pl.program_id(0)
    nsteps = pl.num_programs(0)
    nchunk = o_ref.shape[2] // _BC
    j = i // nchunk
    bb = jax.lax.rem(i, nchunk)
    slot = jax.lax.rem(i, _NBUF)

    @pl.when(i >= _NBUF)
    def _():
        pltpu.make_async_copy(
            buf.at[slot], o_ref.at[0, :, pl.ds(0, _BC)], sem.at[slot]
        ).wait()

    iota = jax.lax.broadcasted_iota(jnp.int32, (_NC, _BC), 0)
    buf[slot] = (xt_ref[0] == iota).astype(jnp.float32)
    pltpu.make_async_copy(
        buf.at[slot], o_ref.at[j, :, pl.ds(bb * _BC, _BC)], sem.at[slot]
    ).start()

    @pl.when(i == nsteps - 1)
    def _():
        for k in range(_NBUF):
            pltpu.make_async_copy(
                buf.at[k], o_ref.at[0, :, pl.ds(0, _BC)], sem.at[k]
            ).wait()


def kernel(x):
    B, S = x.shape
    nchunk = B // _BC
    xt = x.T.reshape(S * nchunk, 1, _BC)
    yt = pl.pallas_call(
        _onehot_ring,
        grid=(S * nchunk,),
        in_specs=[pl.BlockSpec((1, 1, _BC), lambda i: (i, 0, 0))],
        out_specs=pl.BlockSpec(memory_space=pl.ANY),
        out_shape=jax.ShapeDtypeStruct((S, _NC, B), jnp.float32),
        scratch_shapes=[
            pltpu.VMEM((_NBUF, _NC, _BC), jnp.float32),
            pltpu.SemaphoreType.DMA((_NBUF,)),
        ],
    )(xt)
    return yt.transpose(2, 0, 1)
